# Initial kernel scaffold; baseline (speedup 1.0000x reference)
import functools

import jax
import jax.numpy as jnp
from jax import lax
from jax.experimental import pallas as pl
from jax.experimental.pallas import tpu as pltpu
from jax.experimental.pallas import tpu_sc as plsc

_NC = 2
_NS = 16
_NW = _NC * _NS
_CHUNK = 128
_LANES = 16


def _mm_bias(x, w, b, rows_per_block=1000):
    n, k = x.shape
    m = w.shape[1]
    grid = n // rows_per_block

    def body(x_ref, w_ref, b_ref, o_ref):
        o_ref[...] = (
            jnp.dot(x_ref[...], w_ref[...], preferred_element_type=jnp.float32)
            + b_ref[...]
        )

    return pl.pallas_call(
        body,
        grid=(grid,),
        in_specs=[
            pl.BlockSpec((rows_per_block, k), lambda i: (i, 0)),
            pl.BlockSpec((k, m), lambda i: (0, 0)),
            pl.BlockSpec((1, m), lambda i: (0, 0)),
        ],
        out_specs=pl.BlockSpec((rows_per_block, m), lambda i: (i, 0)),
        out_shape=jax.ShapeDtypeStruct((n, m), jnp.float32),
    )(x, w, b.reshape(1, m))


def _relu_sum_mm_bias(p, w, b, rows_per_block=1000):
    _, n, k = p.shape
    m = w.shape[1]
    grid = n // rows_per_block

    def body(p_ref, w_ref, b_ref, o_ref):
        h = jnp.maximum(p_ref[0] + p_ref[1], 0.0)
        o_ref[...] = (
            jnp.dot(h, w_ref[...], preferred_element_type=jnp.float32)
            + b_ref[...]
        )

    return pl.pallas_call(
        body,
        grid=(grid,),
        in_specs=[
            pl.BlockSpec((2, rows_per_block, k), lambda i: (0, i, 0)),
            pl.BlockSpec((k, m), lambda i: (0, 0)),
            pl.BlockSpec((1, m), lambda i: (0, 0)),
        ],
        out_specs=pl.BlockSpec((rows_per_block, m), lambda i: (i, 0)),
        out_shape=jax.ShapeDtypeStruct((n, m), jnp.float32),
    )(p, w, b.reshape(1, m))


@functools.lru_cache(maxsize=None)
def _make_agg(n, d, e_pad):
    per_w = e_pad // _NW
    n_chunks = per_w // _CHUNK
    rows_per_tile = n // _NS
    d_vecs = d // _LANES

    pieces = []
    off = 0
    while off < rows_per_tile:
        sz = min(_CHUNK, rows_per_tile - off)
        pieces.append((off, sz))
        off += sz

    mesh = plsc.VectorSubcoreMesh(core_axis_name="c", subcore_axis_name="s")

    @functools.partial(
        pl.kernel,
        out_type=jax.ShapeDtypeStruct((_NC, n, d), jnp.float32),
        mesh=mesh,
        scratch_types=[
            pltpu.VMEM((_CHUNK,), jnp.int32),
            pltpu.VMEM((_CHUNK,), jnp.int32),
            pltpu.VMEM((_CHUNK,), jnp.float32),
            pltpu.VMEM((_CHUNK, d), jnp.float32),
            pltpu.VMEM_SHARED((n, d), jnp.float32),
            pltpu.SemaphoreType.DMA,
        ],
    )
    def agg(hw_hbm, src_hbm, dst_hbm, ew_hbm, out_hbm,
            src_v, dst_v, ew_v, rows_v, acc_sh, sem):
        cid = lax.axis_index("c")
        sid = lax.axis_index("s")
        wid = sid * _NC + cid

        def zero_row(i, carry):
            for j in range(d_vecs):
                rows_v[i, pl.ds(j * _LANES, _LANES)] = jnp.zeros(
                    (_LANES,), jnp.float32)
            return carry
        lax.fori_loop(0, _CHUNK, zero_row, 0)
        row0 = sid * rows_per_tile
        for poff, psz in pieces:
            pltpu.sync_copy(rows_v.at[pl.ds(0, psz)],
                            acc_sh.at[pl.ds(row0 + poff, psz)])
        plsc.subcore_barrier()

        ebase = wid * per_w

        def chunk(g, carry):
            off = ebase + g * _CHUNK
            pltpu.sync_copy(src_hbm.at[pl.ds(off, _CHUNK)], src_v)
            pltpu.sync_copy(ew_hbm.at[pl.ds(off, _CHUNK)], ew_v)
            pltpu.async_copy(hw_hbm.at[src_v], rows_v, sem).wait()

            def scale_row(i, c2):
                w = ew_v[i]
                for j in range(d_vecs):
                    sl = pl.ds(j * _LANES, _LANES)
                    rows_v[i, sl] = rows_v[i, sl] * w
                return c2
            lax.fori_loop(0, _CHUNK, scale_row, 0)

            pltpu.sync_copy(dst_hbm.at[pl.ds(off, _CHUNK)], dst_v)
            pltpu.sync_copy(rows_v, acc_sh.at[dst_v], add=True)
            return carry
        lax.fori_loop(0, n_chunks, chunk, 0)
        plsc.subcore_barrier()

        for poff, psz in pieces:
            pltpu.sync_copy(acc_sh.at[pl.ds(row0 + poff, psz)],
                            out_hbm.at[cid, pl.ds(row0 + poff, psz)])

    return agg


def kernel(feats, edge_index, edge_weight, W1, b1, W2, b2, Wmu, bmu, Wlv, blv):
    n, d = feats.shape
    e = edge_index.shape[1]
    e_pad = -(-e // (_NW * _CHUNK)) * (_NW * _CHUNK)

    src = jnp.pad(edge_index[0], (0, e_pad - e))
    dst = jnp.pad(edge_index[1], (0, e_pad - e))
    ew = jnp.pad(edge_weight, (0, e_pad - e))

    agg = _make_agg(n, d, e_pad)

    hw1 = _mm_bias(feats, W1, b1)
    p1 = agg(hw1, src, dst, ew)
    hw2 = _relu_sum_mm_bias(p1, W2, b2)
    p2 = agg(hw2, src, dst, ew)
    wcat = jnp.concatenate([Wmu, Wlv], axis=1)
    bcat = jnp.concatenate([bmu, blv])
    mv = _relu_sum_mm_bias(p2, wcat, bcat)
    l = Wmu.shape[1]
    return mv[:, :l], mv[:, l:]


# speedup vs baseline: 3.8637x; 3.8637x over previous
import functools

import jax
import jax.numpy as jnp
from jax import lax
from jax.experimental import pallas as pl
from jax.experimental.pallas import tpu as pltpu
from jax.experimental.pallas import tpu_sc as plsc

_NC = 2
_NS = 16
_NW = _NC * _NS
_CHUNK = 128
_LANES = 16


def _mm_bias(x, w, b, rows_per_block=1000):
    n, k = x.shape
    m = w.shape[1]
    grid = n // rows_per_block

    def body(x_ref, w_ref, b_ref, o_ref):
        o_ref[...] = (
            jnp.dot(x_ref[...], w_ref[...], preferred_element_type=jnp.float32)
            + b_ref[...]
        )

    return pl.pallas_call(
        body,
        grid=(grid,),
        in_specs=[
            pl.BlockSpec((rows_per_block, k), lambda i: (i, 0)),
            pl.BlockSpec((k, m), lambda i: (0, 0)),
            pl.BlockSpec((1, m), lambda i: (0, 0)),
        ],
        out_specs=pl.BlockSpec((rows_per_block, m), lambda i: (i, 0)),
        out_shape=jax.ShapeDtypeStruct((n, m), jnp.float32),
    )(x, w, b.reshape(1, m))


def _relu_sum_mm_bias(p, w, b, n, rows_per_block=1000):
    k = p.shape[2]
    m = w.shape[1]
    grid = n // rows_per_block

    def body(p_ref, w_ref, b_ref, o_ref):
        h = jnp.maximum(p_ref[0] + p_ref[1], 0.0)
        o_ref[...] = (
            jnp.dot(h, w_ref[...], preferred_element_type=jnp.float32)
            + b_ref[...]
        )

    return pl.pallas_call(
        body,
        grid=(grid,),
        in_specs=[
            pl.BlockSpec((2, rows_per_block, k), lambda i: (0, i, 0)),
            pl.BlockSpec((k, m), lambda i: (0, 0)),
            pl.BlockSpec((1, m), lambda i: (0, 0)),
        ],
        out_specs=pl.BlockSpec((rows_per_block, m), lambda i: (i, 0)),
        out_shape=jax.ShapeDtypeStruct((n, m), jnp.float32),
    )(p, w, b.reshape(1, m))


@functools.lru_cache(maxsize=None)
def _make_agg(n, d, e_pad):
    per_w = e_pad // _NW
    n_chunks = per_w // _CHUNK
    rows_per_tile = -(-n // (_NS * 8)) * 8
    n_pad = rows_per_tile * _NS
    d_vecs = d // _LANES

    pieces = []
    off = 0
    while off < rows_per_tile:
        sz = min(_CHUNK, rows_per_tile - off)
        pieces.append((off, sz))
        off += sz

    mesh = plsc.VectorSubcoreMesh(core_axis_name="c", subcore_axis_name="s")

    @functools.partial(
        pl.kernel,
        out_type=jax.ShapeDtypeStruct((_NC, n_pad, d), jnp.float32),
        mesh=mesh,
        scratch_types=[
            pltpu.VMEM((_CHUNK,), jnp.int32),
            pltpu.VMEM((_CHUNK,), jnp.int32),
            pltpu.VMEM((_CHUNK,), jnp.float32),
            pltpu.VMEM((_CHUNK, d), jnp.float32),
            pltpu.VMEM_SHARED((n_pad, d), jnp.float32),
            pltpu.SemaphoreType.DMA,
        ],
    )
    def agg(hw_hbm, src_hbm, dst_hbm, ew_hbm, out_hbm,
            src_v, dst_v, ew_v, rows_v, acc_sh, sem):
        cid = lax.axis_index("c")
        sid = lax.axis_index("s")
        wid = sid * _NC + cid

        def zero_row(i, carry):
            for j in range(d_vecs):
                rows_v[i, pl.ds(j * _LANES, _LANES)] = jnp.zeros(
                    (_LANES,), jnp.float32)
            return carry
        lax.fori_loop(0, _CHUNK, zero_row, 0)
        row0 = sid * rows_per_tile
        for poff, psz in pieces:
            pltpu.sync_copy(rows_v.at[pl.ds(0, psz)],
                            acc_sh.at[pl.ds(row0 + poff, psz)])
        plsc.subcore_barrier()

        ebase = wid * per_w

        def chunk(g, carry):
            off = ebase + g * _CHUNK
            pltpu.sync_copy(src_hbm.at[pl.ds(off, _CHUNK)], src_v)
            pltpu.sync_copy(ew_hbm.at[pl.ds(off, _CHUNK)], ew_v)
            pltpu.async_copy(hw_hbm.at[src_v], rows_v, sem).wait()

            def scale_grp(g, c2):
                wv = ew_v[pl.ds(g * _LANES, _LANES)]
                for lane in range(_LANES):
                    w = wv[lane]
                    for j in range(d_vecs):
                        sl = pl.ds(j * _LANES, _LANES)
                        rows_v[g * _LANES + lane, sl] = (
                            rows_v[g * _LANES + lane, sl] * w)
                return c2
            lax.fori_loop(0, _CHUNK // _LANES, scale_grp, 0)

            pltpu.sync_copy(dst_hbm.at[pl.ds(off, _CHUNK)], dst_v)
            pltpu.sync_copy(rows_v, acc_sh.at[dst_v], add=True)
            return carry
        lax.fori_loop(0, n_chunks, chunk, 0)
        plsc.subcore_barrier()

        for poff, psz in pieces:
            pltpu.sync_copy(acc_sh.at[pl.ds(row0 + poff, psz)],
                            out_hbm.at[cid, pl.ds(row0 + poff, psz)])

    return agg


def kernel(feats, edge_index, edge_weight, W1, b1, W2, b2, Wmu, bmu, Wlv, blv):
    n, d = feats.shape
    e = edge_index.shape[1]
    e_pad = -(-e // (_NW * _CHUNK)) * (_NW * _CHUNK)

    src = jnp.pad(edge_index[0], (0, e_pad - e))
    dst = jnp.pad(edge_index[1], (0, e_pad - e))
    ew = jnp.pad(edge_weight, (0, e_pad - e))

    agg = _make_agg(n, d, e_pad)

    hw1 = _mm_bias(feats, W1, b1)
    p1 = agg(hw1, src, dst, ew)
    hw2 = _relu_sum_mm_bias(p1, W2, b2, n)
    p2 = agg(hw2, src, dst, ew)
    wcat = jnp.concatenate([Wmu, Wlv], axis=1)
    bcat = jnp.concatenate([bmu, blv])
    mv = _relu_sum_mm_bias(p2, wcat, bcat, n)
    l = Wmu.shape[1]
    return mv[:, :l], mv[:, l:]
